# baseline (speedup 1.0000x reference)
import functools

import jax
import jax.numpy as jnp
from jax import lax
from jax.experimental import pallas as pl
from jax.experimental.pallas import tpu as pltpu
from jax.experimental.pallas import tpu_sc as plsc

NC = 2
NS = 16
L = 16
UNROLL = 8


def _sc_edge_kernel(E, N, D, C, SS):
    NW = NC * NS
    EPW = E // NW
    NCH = EPW // C
    NSUP = NCH // SS
    G = C // L
    SC_ = SS * C

    mesh = plsc.VectorSubcoreMesh(
        core_axis_name="c", subcore_axis_name="s", num_cores=NC, num_subcores=NS
    )

    @functools.partial(
        pl.kernel,
        out_type=jax.ShapeDtypeStruct((2 * E,), jnp.float32),
        mesh=mesh,
        compiler_params=pltpu.CompilerParams(
            needs_layout_passes=False, use_tc_tiling_on_sc=False),
        scratch_types=[
            pltpu.VMEM((SC_,), jnp.int32),
            pltpu.VMEM((SC_,), jnp.int32),
            pltpu.VMEM((SC_,), jnp.int32),
            pltpu.VMEM((SC_,), jnp.int32),
            pltpu.VMEM((C, D // 2), jnp.int32),
            pltpu.VMEM((C, D // 2), jnp.int32),
            pltpu.VMEM((C, D // 2), jnp.int32),
            pltpu.VMEM((C, D // 2), jnp.int32),
            pltpu.VMEM((SC_,), jnp.float32),
            pltpu.VMEM((SC_,), jnp.float32),
            pltpu.VMEM((SC_,), jnp.float32),
            pltpu.VMEM((SC_,), jnp.float32),
            pltpu.VMEM((N,), jnp.int32),
            pltpu.VMEM((N,), jnp.int32),
            pltpu.VMEM((N,), jnp.float32),
            pltpu.VMEM((N,), jnp.float32),
            pltpu.VMEM((N,), jnp.float32),
            pltpu.VMEM((N,), jnp.float32),
            pltpu.VMEM((N,), jnp.float32),
            pltpu.VMEM((N,), jnp.float32),
            pltpu.SemaphoreType.DMA,
            pltpu.SemaphoreType.DMA,
            pltpu.SemaphoreType.DMA,
            pltpu.SemaphoreType.DMA,
            pltpu.SemaphoreType.DMA,
            pltpu.SemaphoreType.DMA,
        ],
    )
    def k(src_z, dst_z, ei_hbm,
          nsrc_hbm, ndst_hbm, ss_hbm, sb_hbm, st_hbm, ds_hbm, db_hbm, dt_hbm,
          out_hbm,
          sidx0, sidx1, didx0, didx1, u0, u1, v0, v1,
          loc0, loc1, ssm0, ssm1,
          nsrc_v, ndst_v, ss_v, sb_v, st_v, dsc_v, db_v, dt_v,
          sem_i0, sem_i1, sem_r0, sem_r1, sem_o0, sem_o1):
        wid = lax.axis_index("s") * NC + lax.axis_index("c")
        wbase = wid * EPW

        sidx_b = (sidx0, sidx1)
        didx_b = (didx0, didx1)
        u_b = (u0, u1)
        v_b = (v0, v1)
        loc_b = (loc0, loc1)
        ssm_b = (ssm0, ssm1)
        sem_i = (sem_i0, sem_i1)
        sem_r = (sem_r0, sem_r1)
        sem_o = (sem_o0, sem_o1)

        pltpu.sync_copy(nsrc_hbm, nsrc_v)
        pltpu.sync_copy(ndst_hbm, ndst_v)
        pltpu.sync_copy(ss_hbm, ss_v)
        pltpu.sync_copy(sb_hbm, sb_v)
        pltpu.sync_copy(st_hbm, st_v)
        pltpu.sync_copy(ds_hbm, dsc_v)
        pltpu.sync_copy(db_hbm, db_v)
        pltpu.sync_copy(dt_hbm, dt_v)

        lanes = lax.iota(jnp.int32, L)

        def idx_descs(s, sb):
            base = wbase + s * SC_
            return (
                pltpu.make_async_copy(
                    ei_hbm.at[pl.ds(base, SC_)], sidx_b[sb], sem_i[sb]),
                pltpu.make_async_copy(
                    ei_hbm.at[pl.ds(E + base, SC_)], didx_b[sb], sem_i[sb]),
            )

        def row_descs(p, sb, b):
            return (
                pltpu.make_async_copy(
                    src_z.at[sidx_b[sb].at[pl.ds(p * C, C)]], u_b[b], sem_r[b]),
                pltpu.make_async_copy(
                    dst_z.at[didx_b[sb].at[pl.ds(p * C, C)]], v_b[b], sem_r[b]),
            )

        def out_descs(s, sb):
            base = wbase + s * SC_
            return (
                pltpu.make_async_copy(
                    loc_b[sb], out_hbm.at[pl.ds(base, SC_)], sem_o[sb]),
                pltpu.make_async_copy(
                    ssm_b[sb], out_hbm.at[pl.ds(E + base, SC_)], sem_o[sb]),
            )

        def compute_chunk(p, sb, b):
            def group(g, _):
                pos = lanes + g * L
                sidx = sidx_b[sb][pl.ds(p * C + g * L, L)]
                didx = didx_b[sb][pl.ds(p * C + g * L, L)]
                s_nid = plsc.load_gather(nsrc_v, [sidx])
                d_nid = plsc.load_gather(ndst_v, [didx])
                s_scale = plsc.load_gather(ss_v, [s_nid])
                s_bias = plsc.load_gather(sb_v, [s_nid])
                s_std = plsc.load_gather(st_v, [s_nid])
                d_scale = plsc.load_gather(dsc_v, [d_nid])
                d_bias = plsc.load_gather(db_v, [d_nid])
                d_std = plsc.load_gather(dt_v, [d_nid])

                zero = jnp.zeros((L,), jnp.float32)
                DP = D // 2
                himask = jnp.full((L,), -65536, jnp.int32)

                @plsc.parallel_loop(0, DP, step=UNROLL, carry=(zero, zero, zero, zero))
                def accs(d, carry):
                    a = list(carry)
                    dv = lax.broadcast(d, (L,)) + lanes
                    for j in range(UNROLL):
                        wu = plsc.load_gather(u_b[b], [pos, (dv + j) & (DP - 1)])
                        wv = plsc.load_gather(v_b[b], [pos, (dv + j) & (DP - 1)])
                        ulo = plsc.bitcast(wu << 16, jnp.float32)
                        vlo = plsc.bitcast(wv << 16, jnp.float32)
                        uhi = plsc.bitcast(wu & himask, jnp.float32)
                        vhi = plsc.bitcast(wv & himask, jnp.float32)
                        a[(2 * j) % 4] = a[(2 * j) % 4] + ulo * vlo
                        a[(2 * j + 1) % 4] = a[(2 * j + 1) % 4] + uhi * vhi
                    return tuple(a)

                acc = (accs[0] + accs[1]) + (accs[2] + accs[3])

                loc = s_scale * acc * d_scale + s_bias + d_bias
                loc_b[sb][pl.ds(p * C + g * L, L)] = loc
                x = s_std + d_std
                m = jnp.exp(-jnp.abs(x)) + 1.0
                t = (m - 1.0) / (m + 1.0)
                t2 = t * t
                log_m = 2.0 * t * (1.0 + t2 * (1.0 / 3.0 + t2 * (0.2 + t2 * (1.0 / 7.0))))
                ssm_b[sb][pl.ds(p * C + g * L, L)] = jnp.maximum(x, 0.0) + log_m + 1e-4
                return 0

            lax.fori_loop(0, G, group, 0)

        def super_body(s, sp):
            for p in range(SS):
                b = (sp + p) % 2
                nb = (b + 1) % 2
                if p == 0:
                    @pl.when(s + 1 < NSUP)
                    def _():
                        for dsc in idx_descs(s + 1, (sp + 1) % 2):
                            dsc.start()
                    @pl.when(s >= 2)
                    def _():
                        for dsc in out_descs(s - 2, sp):
                            dsc.wait()
                if p < SS - 1:
                    for dsc in row_descs(p + 1, sp, nb):
                        dsc.start()
                else:
                    @pl.when(s + 1 < NSUP)
                    def _():
                        for dsc in idx_descs(s + 1, (sp + 1) % 2):
                            dsc.wait()
                        for dsc in row_descs(0, (sp + 1) % 2, nb):
                            dsc.start()
                for dsc in row_descs(p, sp, b):
                    dsc.wait()
                compute_chunk(p, sb=sp, b=b)
            for dsc in out_descs(s, sp):
                dsc.start()

        for dsc in idx_descs(0, 0):
            dsc.start()
        for dsc in idx_descs(0, 0):
            dsc.wait()
        for dsc in row_descs(0, 0, 0):
            dsc.start()

        def outer(s, _):
            @pl.when(s % 2 == 0)
            def _():
                super_body(s, 0)

            @pl.when(s % 2 == 1)
            def _():
                super_body(s, 1)

            return 0

        lax.fori_loop(0, NSUP, outer, 0)

        for dsc in out_descs(NSUP - 2, (NSUP - 2) % 2):
            dsc.wait()
        for dsc in out_descs(NSUP - 1, (NSUP - 1) % 2):
            dsc.wait()

    return k


def kernel(src_z, dst_z, edge_index, n_id_src, n_id_dst,
           src_scale_table, src_bias_table, src_std_table,
           dst_scale_table, dst_bias_table, dst_std_table):
    N, D = src_z.shape
    E = edge_index.shape[1]
    C = 80
    SS = 5
    assert E % (NC * NS * C * SS) == 0 and D % L == 0

    ei_flat = edge_index.reshape(2 * E)
    src_zp = lax.bitcast_convert_type(
        src_z.astype(jnp.bfloat16).reshape(N, D // 2, 2), jnp.int32)
    dst_zp = lax.bitcast_convert_type(
        dst_z.astype(jnp.bfloat16).reshape(N, D // 2, 2), jnp.int32)

    out = _sc_edge_kernel(E, N, D, C, SS)(
        src_zp, dst_zp, ei_flat, n_id_src, n_id_dst,
        src_scale_table, src_bias_table, src_std_table,
        dst_scale_table, dst_bias_table, dst_std_table)
    return out.reshape(2, E)


# speedup vs baseline: 1.1511x; 1.1511x over previous
import functools

import jax
import jax.numpy as jnp
from jax import lax
from jax.experimental import pallas as pl
from jax.experimental.pallas import tpu as pltpu
from jax.experimental.pallas import tpu_sc as plsc

NC = 2
NS = 16
L = 16
UNROLL = 16


def _sc_edge_kernel(E, N, D, C, SS):
    NW = NC * NS
    EPW = E // NW
    NCH = EPW // C
    NSUP = NCH // SS
    G = C // L
    SC_ = SS * C

    mesh = plsc.VectorSubcoreMesh(
        core_axis_name="c", subcore_axis_name="s", num_cores=NC, num_subcores=NS
    )

    @functools.partial(
        pl.kernel,
        out_type=jax.ShapeDtypeStruct((2 * E,), jnp.float32),
        mesh=mesh,
        compiler_params=pltpu.CompilerParams(needs_layout_passes=False),
        scratch_types=[
            pltpu.VMEM((SC_,), jnp.int32),
            pltpu.VMEM((SC_,), jnp.int32),
            pltpu.VMEM((SC_,), jnp.int32),
            pltpu.VMEM((SC_,), jnp.int32),
            pltpu.VMEM((C, D), jnp.float32),
            pltpu.VMEM((C, D), jnp.float32),
            pltpu.VMEM((C, D), jnp.float32),
            pltpu.VMEM((C, D), jnp.float32),
            pltpu.VMEM((SC_,), jnp.float32),
            pltpu.VMEM((SC_,), jnp.float32),
            pltpu.VMEM((SC_,), jnp.float32),
            pltpu.VMEM((SC_,), jnp.float32),
            pltpu.VMEM((N,), jnp.int32),
            pltpu.VMEM((N,), jnp.int32),
            pltpu.VMEM((N,), jnp.float32),
            pltpu.VMEM((N,), jnp.float32),
            pltpu.VMEM((N,), jnp.float32),
            pltpu.VMEM((N,), jnp.float32),
            pltpu.VMEM((N,), jnp.float32),
            pltpu.VMEM((N,), jnp.float32),
            pltpu.SemaphoreType.DMA,
            pltpu.SemaphoreType.DMA,
            pltpu.SemaphoreType.DMA,
            pltpu.SemaphoreType.DMA,
            pltpu.SemaphoreType.DMA,
            pltpu.SemaphoreType.DMA,
        ],
    )
    def k(src_z, dst_z, ei_hbm,
          nsrc_hbm, ndst_hbm, ss_hbm, sb_hbm, st_hbm, ds_hbm, db_hbm, dt_hbm,
          out_hbm,
          sidx0, sidx1, didx0, didx1, u0, u1, v0, v1,
          loc0, loc1, ssm0, ssm1,
          nsrc_v, ndst_v, ss_v, sb_v, st_v, dsc_v, db_v, dt_v,
          sem_i0, sem_i1, sem_r0, sem_r1, sem_o0, sem_o1):
        wid = lax.axis_index("s") * NC + lax.axis_index("c")
        wbase = wid * EPW

        sidx_b = (sidx0, sidx1)
        didx_b = (didx0, didx1)
        u_b = (u0, u1)
        v_b = (v0, v1)
        loc_b = (loc0, loc1)
        ssm_b = (ssm0, ssm1)
        sem_i = (sem_i0, sem_i1)
        sem_r = (sem_r0, sem_r1)
        sem_o = (sem_o0, sem_o1)

        table_copies = [
            (nsrc_hbm, nsrc_v), (ndst_hbm, ndst_v),
            (ss_hbm, ss_v), (sb_hbm, sb_v), (st_hbm, st_v),
            (ds_hbm, dsc_v), (db_hbm, db_v), (dt_hbm, dt_v),
        ]
        for src_r, dst_r in table_copies:
            pltpu.make_async_copy(src_r, dst_r, sem_r0).start()
        for src_r, dst_r in table_copies:
            pltpu.make_async_copy(src_r, dst_r, sem_r0).wait()

        lanes = lax.iota(jnp.int32, L)

        def idx_descs(s, sb):
            base = wbase + s * SC_
            return (
                pltpu.make_async_copy(
                    ei_hbm.at[pl.ds(base, SC_)], sidx_b[sb], sem_i[sb]),
                pltpu.make_async_copy(
                    ei_hbm.at[pl.ds(E + base, SC_)], didx_b[sb], sem_i[sb]),
            )

        def row_descs(p, sb, b):
            return (
                pltpu.make_async_copy(
                    src_z.at[sidx_b[sb].at[pl.ds(p * C, C)]], u_b[b], sem_r[b]),
                pltpu.make_async_copy(
                    dst_z.at[didx_b[sb].at[pl.ds(p * C, C)]], v_b[b], sem_r[b]),
            )

        def out_descs(s, sb):
            base = wbase + s * SC_
            return (
                pltpu.make_async_copy(
                    loc_b[sb], out_hbm.at[pl.ds(base, SC_)], sem_o[sb]),
                pltpu.make_async_copy(
                    ssm_b[sb], out_hbm.at[pl.ds(E + base, SC_)], sem_o[sb]),
            )

        def compute_chunk(p, sb, b):
            def group(g, _):
                pos = lanes + g * L
                sidx = sidx_b[sb][pl.ds(p * C + g * L, L)]
                didx = didx_b[sb][pl.ds(p * C + g * L, L)]
                s_nid = plsc.load_gather(nsrc_v, [sidx])
                d_nid = plsc.load_gather(ndst_v, [didx])
                s_scale = plsc.load_gather(ss_v, [s_nid])
                s_bias = plsc.load_gather(sb_v, [s_nid])
                s_std = plsc.load_gather(st_v, [s_nid])
                d_scale = plsc.load_gather(dsc_v, [d_nid])
                d_bias = plsc.load_gather(db_v, [d_nid])
                d_std = plsc.load_gather(dt_v, [d_nid])

                zero = jnp.zeros((L,), jnp.float32)

                @plsc.parallel_loop(0, D, step=UNROLL, carry=(zero, zero, zero, zero))
                def accs(d, carry):
                    a = list(carry)
                    dv = lax.broadcast(d, (L,)) + lanes
                    for j in range(UNROLL):
                        uu = plsc.load_gather(u_b[b], [pos, (dv + j) & (D - 1)])
                        vv = plsc.load_gather(v_b[b], [pos, (dv + j) & (D - 1)])
                        a[j % 4] = a[j % 4] + uu * vv
                    return tuple(a)

                acc = (accs[0] + accs[1]) + (accs[2] + accs[3])

                loc = s_scale * acc * d_scale + s_bias + d_bias
                loc_b[sb][pl.ds(p * C + g * L, L)] = loc
                x = s_std + d_std
                m = jnp.exp(-jnp.abs(x)) + 1.0
                t = (m - 1.0) / (m + 1.0)
                t2 = t * t
                log_m = 2.0 * t * (1.0 + t2 * (1.0 / 3.0 + t2 * (0.2 + t2 * (1.0 / 7.0))))
                ssm_b[sb][pl.ds(p * C + g * L, L)] = jnp.maximum(x, 0.0) + log_m + 1e-4
                return 0

            lax.fori_loop(0, G, group, 0)

        def super_body(s, sp):
            for p in range(SS):
                b = (sp + p) % 2
                nb = (b + 1) % 2
                if p == 0:
                    @pl.when(s + 1 < NSUP)
                    def _():
                        for dsc in idx_descs(s + 1, (sp + 1) % 2):
                            dsc.start()
                    @pl.when(s >= 2)
                    def _():
                        for dsc in out_descs(s - 2, sp):
                            dsc.wait()
                if p < SS - 1:
                    for dsc in row_descs(p + 1, sp, nb):
                        dsc.start()
                else:
                    @pl.when(s + 1 < NSUP)
                    def _():
                        for dsc in idx_descs(s + 1, (sp + 1) % 2):
                            dsc.wait()
                        for dsc in row_descs(0, (sp + 1) % 2, nb):
                            dsc.start()
                for dsc in row_descs(p, sp, b):
                    dsc.wait()
                compute_chunk(p, sb=sp, b=b)
            for dsc in out_descs(s, sp):
                dsc.start()

        for dsc in idx_descs(0, 0):
            dsc.start()
        for dsc in idx_descs(0, 0):
            dsc.wait()
        for dsc in row_descs(0, 0, 0):
            dsc.start()

        def outer(s, _):
            @pl.when(s % 2 == 0)
            def _():
                super_body(s, 0)

            @pl.when(s % 2 == 1)
            def _():
                super_body(s, 1)

            return 0

        lax.fori_loop(0, NSUP, outer, 0)

        for dsc in out_descs(NSUP - 2, (NSUP - 2) % 2):
            dsc.wait()
        for dsc in out_descs(NSUP - 1, (NSUP - 1) % 2):
            dsc.wait()

    return k


def kernel(src_z, dst_z, edge_index, n_id_src, n_id_dst,
           src_scale_table, src_bias_table, src_std_table,
           dst_scale_table, dst_bias_table, dst_std_table):
    N, D = src_z.shape
    E = edge_index.shape[1]
    C = 80
    SS = 5
    assert E % (NC * NS * C * SS) == 0 and D % L == 0

    ei_flat = edge_index.reshape(2 * E)

    out = _sc_edge_kernel(E, N, D, C, SS)(
        src_z, dst_z, ei_flat, n_id_src, n_id_dst,
        src_scale_table, src_bias_table, src_std_table,
        dst_scale_table, dst_bias_table, dst_std_table)
    return out.reshape(2, E)
